# baseline (speedup 1.0000x reference)
import functools

import jax
import jax.numpy as jnp
from jax import lax
from jax.experimental import pallas as pl
from jax.experimental.pallas import tpu as pltpu
from jax.experimental.pallas import tpu_sc as plsc

N = 10000
NPAD = 10240
E = 320000
CHUNK = 128
EPAD = 327680
D = 128
NC, NS = 2, 16
NCHUNKS = EPAD // CHUNK
REAL_CHUNKS = E // CHUNK
CH_W = NCHUNKS // (NC * NS)
CH_A = NCHUNKS // NS
RPT = NPAD // NS
RB = 2000

_mesh = plsc.VectorSubcoreMesh(core_axis_name="c", subcore_axis_name="s")


@functools.partial(
    pl.kernel,
    out_type=jax.ShapeDtypeStruct((NC, NPAD), jnp.float32),
    mesh=_mesh,
    scratch_types=[
        pltpu.VMEM((CH_A, CHUNK), jnp.int32),
        pltpu.VMEM((CHUNK,), jnp.float32),
        pltpu.VMEM((RPT,), jnp.float32),
        pltpu.SemaphoreType.DMA,
        pltpu.VMEM_SHARED((NPAD,), jnp.float32),
    ],
)
def _deg_kernel(eidx_hbm, out_hbm, idx_v, ones_v, zbuf_v, dsem, deg_sh):
    c = lax.axis_index("c")
    s = lax.axis_index("s")

    @pl.loop(0, RPT, step=16)
    def _(i):
        zbuf_v[pl.ds(i, 16)] = jnp.zeros((16,), jnp.float32)

    @pl.loop(0, CHUNK, step=16)
    def _(i):
        ones_v[pl.ds(i, 16)] = jnp.ones((16,), jnp.float32)

    pltpu.sync_copy(zbuf_v, deg_sh.at[pl.ds(s * RPT, RPT)])
    pltpu.sync_copy(eidx_hbm.at[c, pl.ds(s * CH_A, CH_A), :], idx_v)
    plsc.subcore_barrier()

    @pl.loop(0, CH_A, step=8)
    def _(j):
        for k in range(8):
            @pl.when(s * CH_A + j + k < REAL_CHUNKS)
            def _():
                pltpu.async_copy(ones_v, deg_sh.at[idx_v.at[j + k]], dsem,
                                 add=True)
        for k in range(8):
            @pl.when(s * CH_A + j + k < REAL_CHUNKS)
            def _():
                pltpu.make_async_copy(ones_v, deg_sh.at[idx_v.at[0]],
                                      dsem).wait()

    plsc.subcore_barrier()
    pltpu.sync_copy(deg_sh.at[pl.ds(s * RPT, RPT)],
                    out_hbm.at[c, pl.ds(s * RPT, RPT)])


HALF = CH_W // 2


@functools.partial(
    pl.kernel,
    out_type=jax.ShapeDtypeStruct((NC, NPAD, D), jnp.float32),
    mesh=_mesh,
    scratch_types=[
        pltpu.VMEM((HALF, CHUNK), jnp.int32),
        pltpu.VMEM((HALF, CHUNK), jnp.int32),
        pltpu.VMEM((CHUNK, D), jnp.float32),
        pltpu.VMEM((CHUNK, D), jnp.float32),
        pltpu.SemaphoreType.DMA,
        pltpu.SemaphoreType.DMA,
        pltpu.VMEM_SHARED((NPAD, D), jnp.float32),
    ],
)
def _conv_kernel(h_hbm, eidx_hbm, out_hbm, src_v, dst_v, buf0, buf1,
                 gsem0, gsem1, agg_sh):
    c = lax.axis_index("c")
    s = lax.axis_index("s")
    w = c * NS + s
    base = s * RPT

    bufs = (buf0, buf1)
    gsems = (gsem0, gsem1)

    def start_gather(jj, b):
        pltpu.async_copy(h_hbm.at[src_v.at[jj]], bufs[b], gsems[b])

    def wait_gather(b):
        pltpu.make_async_copy(h_hbm.at[src_v.at[0]], bufs[b], gsems[b]).wait()

    def scat_sync(jj, b):
        pltpu.sync_copy(bufs[b], agg_sh.at[dst_v.at[jj]], add=True)

    def load_idx(h):
        pltpu.sync_copy(eidx_hbm.at[0, pl.ds((w * 2 + h) * HALF, HALF), :],
                        src_v)
        pltpu.sync_copy(eidx_hbm.at[1, pl.ds((w * 2 + h) * HALF, HALF), :],
                        dst_v)

    @pl.loop(0, CHUNK)
    def _(i):
        @pl.loop(0, D, step=16)
        def _(l):
            buf1[i, pl.ds(l, 16)] = jnp.zeros((16,), jnp.float32)

    for k in range(RPT // CHUNK):
        pltpu.async_copy(buf1, agg_sh.at[pl.ds(base + k * CHUNK, CHUNK), :],
                         gsem1)
    load_idx(0)
    start_gather(0, 0)
    for k in range(RPT // CHUNK):
        pltpu.make_async_copy(buf1, agg_sh.at[pl.ds(base, CHUNK), :],
                              gsem1).wait()
    plsc.subcore_barrier()

    for h in range(2):
        if h:
            load_idx(h)
            start_gather(0, 0)
        @pl.loop(0, HALF - 2, step=2)
        def _(j):
            wait_gather(0)
            start_gather(j + 1, 1)
            scat_sync(j, 0)
            wait_gather(1)
            start_gather(j + 2, 0)
            scat_sync(j + 1, 1)

        wait_gather(0)
        start_gather(HALF - 1, 1)
        scat_sync(HALF - 2, 0)
        wait_gather(1)
        scat_sync(HALF - 1, 1)

    plsc.subcore_barrier()
    pltpu.sync_copy(agg_sh.at[pl.ds(s * RPT, RPT), :],
                    out_hbm.at[c, pl.ds(s * RPT, RPT), :])


def _mm_body(x_ref, w_ref, o_ref):
    o_ref[...] = jnp.dot(x_ref[...], w_ref[...],
                         preferred_element_type=jnp.float32)


def _scale_body(deg_ref, m_ref, o_ref):
    norm = lax.rsqrt(jnp.maximum(deg_ref[...], 1.0))
    o_ref[...] = norm * m_ref[...]


def _mid_body(p0_ref, p1_ref, degi_ref, dego_ref, b1_ref, w_ref, o_ref):
    ndst = lax.rsqrt(jnp.maximum(degi_ref[...], 1.0))
    t = (p0_ref[0] + p1_ref[0]) * ndst + b1_ref[...]
    h = jnp.maximum(t, 0.0)
    nsrc = lax.rsqrt(jnp.maximum(dego_ref[...], 1.0))
    o_ref[...] = nsrc * jnp.dot(h, w_ref[...],
                                preferred_element_type=jnp.float32)


def _out_body(q0_ref, q1_ref, degi_ref, b2_ref, o_ref):
    ndst = lax.rsqrt(jnp.maximum(degi_ref[...], 1.0))
    y = (q0_ref[0] + q1_ref[0]) * ndst + b2_ref[...]
    m = jnp.max(y, axis=1, keepdims=True)
    lse = jnp.log(jnp.sum(jnp.exp(y - m), axis=1, keepdims=True))
    o_ref[...] = y - m - lse


_GRID = N // RB

_mm = pl.pallas_call(
    _mm_body,
    grid=(_GRID,),
    in_specs=[
        pl.BlockSpec((RB, D), lambda i: (i, 0)),
        pl.BlockSpec((D, D), lambda i: (0, 0)),
    ],
    out_specs=pl.BlockSpec((RB, D), lambda i: (i, 0)),
    out_shape=jax.ShapeDtypeStruct((N, D), jnp.float32),
)

_scale = pl.pallas_call(
    _scale_body,
    grid=(_GRID,),
    in_specs=[
        pl.BlockSpec((RB, 1), lambda i: (i, 0)),
        pl.BlockSpec((RB, D), lambda i: (i, 0)),
    ],
    out_specs=pl.BlockSpec((RB, D), lambda i: (i, 0)),
    out_shape=jax.ShapeDtypeStruct((N, D), jnp.float32),
)

_mid = pl.pallas_call(
    _mid_body,
    grid=(_GRID,),
    in_specs=[
        pl.BlockSpec((1, RB, D), lambda i: (0, i, 0)),
        pl.BlockSpec((1, RB, D), lambda i: (1, i, 0)),
        pl.BlockSpec((RB, 1), lambda i: (i, 0)),
        pl.BlockSpec((RB, 1), lambda i: (i, 0)),
        pl.BlockSpec((1, D), lambda i: (0, 0)),
        pl.BlockSpec((D, D), lambda i: (0, 0)),
    ],
    out_specs=pl.BlockSpec((RB, D), lambda i: (i, 0)),
    out_shape=jax.ShapeDtypeStruct((N, D), jnp.float32),
)

_out = pl.pallas_call(
    _out_body,
    grid=(_GRID,),
    in_specs=[
        pl.BlockSpec((1, RB, D), lambda i: (0, i, 0)),
        pl.BlockSpec((1, RB, D), lambda i: (1, i, 0)),
        pl.BlockSpec((RB, 1), lambda i: (i, 0)),
        pl.BlockSpec((1, D), lambda i: (0, 0)),
    ],
    out_specs=pl.BlockSpec((RB, D), lambda i: (i, 0)),
    out_shape=jax.ShapeDtypeStruct((N, D), jnp.float32),
)


def kernel(x, edge_index, W1, b1, W2, b2):
    ei = edge_index.astype(jnp.int32)
    npad_e = EPAD - E
    r = jnp.arange(npad_e, dtype=jnp.int32)
    pad_src = (r * 131) % N
    pad_dst = N + (r % (NPAD - N))
    ei_pad = jnp.concatenate(
        [ei, jnp.stack([pad_src, pad_dst])], axis=1
    ).reshape(2, NCHUNKS, CHUNK)

    m1 = _mm(x, W1)
    deg = _deg_kernel(ei_pad)
    deg_out = deg[0, :N].reshape(N, 1)
    deg_in = deg[1, :N].reshape(N, 1)

    h1 = _scale(deg_out, m1)
    p = _conv_kernel(h1, ei_pad)
    h2 = _mid(p, p, deg_in, deg_out, b1.reshape(1, D), W2)
    q = _conv_kernel(h2, ei_pad)
    return _out(q, q, deg_in, b2.reshape(1, D))


# speedup vs baseline: 1.0133x; 1.0133x over previous
import functools

import jax
import jax.numpy as jnp
from jax import lax
from jax.experimental import pallas as pl
from jax.experimental.pallas import tpu as pltpu
from jax.experimental.pallas import tpu_sc as plsc

N = 10000
NPAD = 10240
E = 320000
CHUNK = 128
EPAD = 327680
D = 128
NC, NS = 2, 16
NCHUNKS = EPAD // CHUNK
REAL_CHUNKS = E // CHUNK
CH_W = NCHUNKS // (NC * NS)
CH_A = NCHUNKS // NS
RPT = NPAD // NS
RB = 2000

_mesh = plsc.VectorSubcoreMesh(core_axis_name="c", subcore_axis_name="s")


@functools.partial(
    pl.kernel,
    out_type=jax.ShapeDtypeStruct((NC, NPAD), jnp.float32),
    mesh=_mesh,
    scratch_types=[
        pltpu.VMEM((CH_A, CHUNK), jnp.int32),
        pltpu.VMEM((CHUNK,), jnp.float32),
        pltpu.VMEM((RPT,), jnp.float32),
        pltpu.SemaphoreType.DMA,
        pltpu.VMEM_SHARED((NPAD,), jnp.float32),
    ],
)
def _deg_kernel(eidx_hbm, out_hbm, idx_v, ones_v, zbuf_v, dsem, deg_sh):
    c = lax.axis_index("c")
    s = lax.axis_index("s")

    @pl.loop(0, RPT, step=16)
    def _(i):
        zbuf_v[pl.ds(i, 16)] = jnp.zeros((16,), jnp.float32)

    @pl.loop(0, CHUNK, step=16)
    def _(i):
        ones_v[pl.ds(i, 16)] = jnp.ones((16,), jnp.float32)

    pltpu.sync_copy(zbuf_v, deg_sh.at[pl.ds(s * RPT, RPT)])
    pltpu.sync_copy(eidx_hbm.at[c, pl.ds(s * CH_A, CH_A), :], idx_v)
    plsc.subcore_barrier()

    @pl.loop(0, CH_A, step=8)
    def _(j):
        for k in range(8):
            @pl.when(s * CH_A + j + k < REAL_CHUNKS)
            def _():
                pltpu.async_copy(ones_v, deg_sh.at[idx_v.at[j + k]], dsem,
                                 add=True)
        for k in range(8):
            @pl.when(s * CH_A + j + k < REAL_CHUNKS)
            def _():
                pltpu.make_async_copy(ones_v, deg_sh.at[idx_v.at[0]],
                                      dsem).wait()

    plsc.subcore_barrier()
    pltpu.sync_copy(deg_sh.at[pl.ds(s * RPT, RPT)],
                    out_hbm.at[c, pl.ds(s * RPT, RPT)])


HALF = CH_W // 2


@functools.partial(
    pl.kernel,
    out_type=jax.ShapeDtypeStruct((NC, NPAD, D), jnp.float32),
    mesh=_mesh,
    scratch_types=[
        pltpu.VMEM((HALF, CHUNK), jnp.int32),
        pltpu.VMEM((HALF, CHUNK), jnp.int32),
        pltpu.VMEM((CHUNK, D), jnp.float32),
        pltpu.VMEM((CHUNK, D), jnp.float32),
        pltpu.SemaphoreType.DMA,
        pltpu.SemaphoreType.DMA,
        pltpu.VMEM_SHARED((NPAD, D), jnp.float32),
    ],
)
def _conv_kernel(h_hbm, eidx_hbm, out_hbm, src_v, dst_v, buf0, buf1,
                 gsem0, gsem1, agg_sh):
    c = lax.axis_index("c")
    s = lax.axis_index("s")
    w = c * NS + s
    base = s * RPT

    bufs = (buf0, buf1)
    gsems = (gsem0, gsem1)

    def start_gather(jj, b):
        pltpu.async_copy(h_hbm.at[src_v.at[jj]], bufs[b], gsems[b])

    def wait_gather(b):
        pltpu.make_async_copy(h_hbm.at[src_v.at[0]], bufs[b], gsems[b]).wait()

    def scat_sync(jj, b):
        pltpu.sync_copy(bufs[b], agg_sh.at[dst_v.at[jj]], add=True)

    def load_idx(h):
        pltpu.sync_copy(eidx_hbm.at[0, pl.ds((w * 2 + h) * HALF, HALF), :],
                        src_v)
        pltpu.sync_copy(eidx_hbm.at[1, pl.ds((w * 2 + h) * HALF, HALF), :],
                        dst_v)

    @pl.loop(0, CHUNK)
    def _(i):
        @pl.loop(0, D, step=16)
        def _(l):
            buf1[i, pl.ds(l, 16)] = jnp.zeros((16,), jnp.float32)

    for k in range(RPT // CHUNK):
        pltpu.async_copy(buf1, agg_sh.at[pl.ds(base + k * CHUNK, CHUNK), :],
                         gsem1)
    load_idx(0)
    start_gather(0, 0)
    for k in range(RPT // CHUNK):
        pltpu.make_async_copy(buf1, agg_sh.at[pl.ds(base, CHUNK), :],
                              gsem1).wait()
    plsc.subcore_barrier()

    for h in range(2):
        if h:
            load_idx(h)
            start_gather(0, 0)
        @pl.loop(0, HALF - 2, step=2)
        def _(j):
            wait_gather(0)
            start_gather(j + 1, 1)
            scat_sync(j, 0)
            wait_gather(1)
            start_gather(j + 2, 0)
            scat_sync(j + 1, 1)

        wait_gather(0)
        start_gather(HALF - 1, 1)
        scat_sync(HALF - 2, 0)
        wait_gather(1)
        scat_sync(HALF - 1, 1)

    plsc.subcore_barrier()
    pltpu.sync_copy(agg_sh.at[pl.ds(s * RPT, RPT), :],
                    out_hbm.at[c, pl.ds(s * RPT, RPT), :])


def _mm_norm_body(deg_ref, x_ref, w_ref, o_ref):
    norm = lax.rsqrt(jnp.maximum(deg_ref[...], 1.0))
    o_ref[...] = norm * jnp.dot(x_ref[...], w_ref[...],
                                preferred_element_type=jnp.float32)


def _mid_body(p0_ref, p1_ref, degi_ref, dego_ref, b1_ref, w_ref, o_ref):
    ndst = lax.rsqrt(jnp.maximum(degi_ref[...], 1.0))
    t = (p0_ref[0] + p1_ref[0]) * ndst + b1_ref[...]
    h = jnp.maximum(t, 0.0)
    nsrc = lax.rsqrt(jnp.maximum(dego_ref[...], 1.0))
    o_ref[...] = nsrc * jnp.dot(h, w_ref[...],
                                preferred_element_type=jnp.float32)


def _out_body(q0_ref, q1_ref, degi_ref, b2_ref, o_ref):
    ndst = lax.rsqrt(jnp.maximum(degi_ref[...], 1.0))
    y = (q0_ref[0] + q1_ref[0]) * ndst + b2_ref[...]
    m = jnp.max(y, axis=1, keepdims=True)
    lse = jnp.log(jnp.sum(jnp.exp(y - m), axis=1, keepdims=True))
    o_ref[...] = y - m - lse


_GRID = N // RB

_mm_norm = pl.pallas_call(
    _mm_norm_body,
    grid=(_GRID,),
    in_specs=[
        pl.BlockSpec((RB, 1), lambda i: (i, 0)),
        pl.BlockSpec((RB, D), lambda i: (i, 0)),
        pl.BlockSpec((D, D), lambda i: (0, 0)),
    ],
    out_specs=pl.BlockSpec((RB, D), lambda i: (i, 0)),
    out_shape=jax.ShapeDtypeStruct((N, D), jnp.float32),
)

_mid = pl.pallas_call(
    _mid_body,
    grid=(_GRID,),
    in_specs=[
        pl.BlockSpec((1, RB, D), lambda i: (0, i, 0)),
        pl.BlockSpec((1, RB, D), lambda i: (1, i, 0)),
        pl.BlockSpec((RB, 1), lambda i: (i, 0)),
        pl.BlockSpec((RB, 1), lambda i: (i, 0)),
        pl.BlockSpec((1, D), lambda i: (0, 0)),
        pl.BlockSpec((D, D), lambda i: (0, 0)),
    ],
    out_specs=pl.BlockSpec((RB, D), lambda i: (i, 0)),
    out_shape=jax.ShapeDtypeStruct((N, D), jnp.float32),
)

_out = pl.pallas_call(
    _out_body,
    grid=(_GRID,),
    in_specs=[
        pl.BlockSpec((1, RB, D), lambda i: (0, i, 0)),
        pl.BlockSpec((1, RB, D), lambda i: (1, i, 0)),
        pl.BlockSpec((RB, 1), lambda i: (i, 0)),
        pl.BlockSpec((1, D), lambda i: (0, 0)),
    ],
    out_specs=pl.BlockSpec((RB, D), lambda i: (i, 0)),
    out_shape=jax.ShapeDtypeStruct((N, D), jnp.float32),
)


def kernel(x, edge_index, W1, b1, W2, b2):
    ei = edge_index.astype(jnp.int32)
    npad_e = EPAD - E
    r = jnp.arange(npad_e, dtype=jnp.int32)
    pad_src = (r * 131) % N
    pad_dst = N + (r % (NPAD - N))
    ei_pad = jnp.concatenate(
        [ei, jnp.stack([pad_src, pad_dst])], axis=1
    ).reshape(2, NCHUNKS, CHUNK)

    deg = _deg_kernel(ei_pad)
    deg_out = deg[0, :N].reshape(N, 1)
    deg_in = deg[1, :N].reshape(N, 1)

    h1 = _mm_norm(deg_out, x, W1)
    p = _conv_kernel(h1, ei_pad)
    h2 = _mid(p, p, deg_in, deg_out, b1.reshape(1, D), W2)
    q = _conv_kernel(h2, ei_pad)
    return _out(q, q, deg_in, b2.reshape(1, D))
